# Initial kernel scaffold; baseline (speedup 1.0000x reference)
import jax
import jax.numpy as jnp
from jax.experimental import pallas as pl


def kernel(x, edge_index, weight, weight_self):
    raise NotImplementedError("write your pallas kernel here")


# speedup vs baseline: 5.6752x; 5.6752x over previous
import functools

import jax
import jax.numpy as jnp
from jax import lax
from jax.experimental import pallas as pl
from jax.experimental.pallas import tpu as pltpu
from jax.experimental.pallas import tpu_sc as plsc

N = 10000
E = 320000
D = 128

NC = 2
NS = 16
NW = NC * NS
EPW = E // NW
CHUNK = 80
NCHUNK = EPW // CHUNK
DTILES = 10
DR = N // DTILES
ZR = 200


def _sc_segment_sum(x, edge_flat):
    mesh = plsc.VectorSubcoreMesh(core_axis_name="c", subcore_axis_name="s")

    @functools.partial(
        pl.kernel,
        mesh=mesh,
        out_type=jax.ShapeDtypeStruct((NC, N, D), jnp.float32),
        scratch_types=[
            pltpu.VMEM((CHUNK,), jnp.int32),
            pltpu.VMEM((CHUNK,), jnp.int32),
            pltpu.VMEM((CHUNK, D), jnp.float32),
            pltpu.VMEM((ZR, D), jnp.float32),
            pltpu.VMEM_SHARED((N, D), jnp.float32),
            pltpu.SemaphoreType.DMA,
        ],
    )
    def body(x_hbm, ei_hbm, out_hbm, src_v, dst_v, rows_v, zero_v, acc_sh, sem):
        c = lax.axis_index("c")
        s = lax.axis_index("s")
        wid = s * NC + c

        def zrow(i, carry):
            def zcol(j, carry2):
                zero_v[i, pl.ds(j * 16, 16)] = jnp.zeros((16,), jnp.float32)
                return carry2
            return lax.fori_loop(0, D // 16, zcol, carry)
        lax.fori_loop(0, ZR, zrow, 0)

        @pl.when(s < DTILES)
        def _():
            for z in range(DR // ZR):
                pltpu.sync_copy(zero_v, acc_sh.at[pl.ds(s * DR + z * ZR, ZR)])
        plsc.subcore_barrier()

        base_w = wid * EPW

        def chunk_body(ci, carry):
            base = base_w + ci * CHUNK
            pltpu.sync_copy(ei_hbm.at[pl.ds(base, CHUNK)], src_v)
            pltpu.sync_copy(ei_hbm.at[pl.ds(E + base, CHUNK)], dst_v)
            pltpu.async_copy(x_hbm.at[src_v], rows_v, sem).wait()
            pltpu.sync_copy(rows_v, acc_sh.at[dst_v], add=True)
            return carry
        lax.fori_loop(0, NCHUNK, chunk_body, 0)

        plsc.subcore_barrier()

        @pl.when(s < DTILES)
        def _():
            pltpu.sync_copy(acc_sh.at[pl.ds(s * DR, DR)],
                            out_hbm.at[c, pl.ds(s * DR, DR)])

    return body(x, edge_flat)


BLK = 1000


def _tc_combine(part, x, weight, weight_self):
    def body(p_ref, x_ref, w_ref, ws_ref, o_ref):
        agg = p_ref[0] + p_ref[1]
        o_ref[...] = (
            jnp.dot(agg, w_ref[...], preferred_element_type=jnp.float32)
            + jnp.dot(x_ref[...], ws_ref[...], preferred_element_type=jnp.float32)
        )

    return pl.pallas_call(
        body,
        grid=(N // BLK,),
        in_specs=[
            pl.BlockSpec((NC, BLK, D), lambda i: (0, i, 0)),
            pl.BlockSpec((BLK, D), lambda i: (i, 0)),
            pl.BlockSpec((D, D), lambda i: (0, 0)),
            pl.BlockSpec((D, D), lambda i: (0, 0)),
        ],
        out_specs=pl.BlockSpec((BLK, D), lambda i: (i, 0)),
        out_shape=jax.ShapeDtypeStruct((N, D), jnp.float32),
    )(part, x, weight, weight_self)


def kernel(x, edge_index, weight, weight_self):
    part = _sc_segment_sum(x, edge_index.reshape(-1))
    return _tc_combine(part, x, weight, weight_self)


# speedup vs baseline: 12.3465x; 2.1755x over previous
import functools

import jax
import jax.numpy as jnp
from jax import lax
from jax.experimental import pallas as pl
from jax.experimental.pallas import tpu as pltpu
from jax.experimental.pallas import tpu_sc as plsc

N = 10000
E = 320000
D = 128

NC = 2
NS = 16
NW = NC * NS
EPW = E // NW
CHUNK = 100
NCHUNK = EPW // CHUNK
DTILES = 10
DR = N // DTILES


def _sc_segment_sum(x, srcs, dsts, zeros):
    mesh = plsc.VectorSubcoreMesh(core_axis_name="c", subcore_axis_name="s")

    @functools.partial(
        pl.kernel,
        mesh=mesh,
        out_type=jax.ShapeDtypeStruct((NC, N, D), jnp.float32),
        scratch_types=[
            pltpu.VMEM((4, CHUNK), jnp.int32),
            pltpu.VMEM((4, CHUNK), jnp.int32),
            pltpu.VMEM((CHUNK, D), jnp.float32),
            pltpu.VMEM((CHUNK, D), jnp.float32),
            pltpu.VMEM_SHARED((N, D), jnp.float32),
            pltpu.SemaphoreType.DMA,
            pltpu.SemaphoreType.DMA,
            pltpu.SemaphoreType.DMA,
            pltpu.SemaphoreType.DMA,
            pltpu.SemaphoreType.DMA,
            pltpu.SemaphoreType.DMA,
        ],
    )
    def body(x_hbm, src_hbm, dst_hbm, zero_hbm, out_hbm, src_v, dst_v,
             rows0, rows1, acc_sh, is0, is1, is2, is3, gsem0, gsem1):
        c = lax.axis_index("c")
        s = lax.axis_index("s")
        wid = s * NC + c

        bufs = (rows0, rows1)
        gsems = (gsem0, gsem1)
        isems = (is0, is1, is2, is3)

        def idx_load(ci, slot):
            pltpu.make_async_copy(
                src_hbm.at[wid, ci], src_v.at[slot], isems[slot]).start()
            pltpu.make_async_copy(
                dst_hbm.at[wid, ci], dst_v.at[slot], isems[slot]).start()

        def idx_wait(slot):
            pltpu.make_async_copy(
                src_hbm.at[wid, 0], src_v.at[slot], isems[slot]).wait()
            pltpu.make_async_copy(
                dst_hbm.at[wid, 0], dst_v.at[slot], isems[slot]).wait()

        def gather_start(ci, b, slot):
            pltpu.make_async_copy(
                x_hbm.at[src_v.at[slot]], bufs[b], gsems[b]).start()

        def gather_wait(b, slot):
            pltpu.make_async_copy(
                x_hbm.at[src_v.at[slot]], bufs[b], gsems[b]).wait()

        for ci in range(4):
            idx_load(ci, ci)

        @pl.when(s < DTILES)
        def _():
            pltpu.sync_copy(zero_hbm.at[pl.ds(s * DR, DR)],
                            acc_sh.at[pl.ds(s * DR, DR)])

        idx_wait(0)
        gather_start(0, 0, 0)
        idx_wait(1)
        gather_start(1, 1, 1)

        plsc.subcore_barrier()

        def step(ci, b, slot):
            gather_wait(b, slot)
            pltpu.sync_copy(bufs[b], acc_sh.at[dst_v.at[slot]], add=True)

            @pl.when(ci + 4 < NCHUNK)
            def _():
                idx_load(ci + 4, slot)

            @pl.when(ci + 2 < NCHUNK)
            def _():
                nslot = (slot + 2) % 4
                idx_wait(nslot)
                gather_start(ci + 2, b, nslot)

        def quad(k, carry):
            ci = 4 * k
            step(ci, 0, 0)
            step(ci + 1, 1, 1)
            step(ci + 2, 0, 2)
            step(ci + 3, 1, 3)
            return carry
        lax.fori_loop(0, NCHUNK // 4, quad, 0)

        plsc.subcore_barrier()

        @pl.when(s < DTILES)
        def _():
            pltpu.sync_copy(acc_sh.at[pl.ds(s * DR, DR)],
                            out_hbm.at[c, pl.ds(s * DR, DR)])

    return body(x, srcs, dsts, zeros)


BLK = 1000


def _tc_combine(part, x, weight, weight_self):
    def body(p_ref, x_ref, w_ref, ws_ref, o_ref):
        agg = p_ref[0] + p_ref[1]
        o_ref[...] = (
            jnp.dot(agg, w_ref[...], preferred_element_type=jnp.float32)
            + jnp.dot(x_ref[...], ws_ref[...], preferred_element_type=jnp.float32)
        )

    return pl.pallas_call(
        body,
        grid=(N // BLK,),
        in_specs=[
            pl.BlockSpec((NC, BLK, D), lambda i: (0, i, 0)),
            pl.BlockSpec((BLK, D), lambda i: (i, 0)),
            pl.BlockSpec((D, D), lambda i: (0, 0)),
            pl.BlockSpec((D, D), lambda i: (0, 0)),
        ],
        out_specs=pl.BlockSpec((BLK, D), lambda i: (i, 0)),
        out_shape=jax.ShapeDtypeStruct((N, D), jnp.float32),
    )(part, x, weight, weight_self)


def kernel(x, edge_index, weight, weight_self):
    srcs = edge_index[0].reshape(NW, NCHUNK, CHUNK)
    dsts = edge_index[1].reshape(NW, NCHUNK, CHUNK)
    zeros = jnp.zeros((N, D), jnp.float32)
    part = _sc_segment_sum(x, srcs, dsts, zeros)
    return _tc_combine(part, x, weight, weight_self)
